# baseline (speedup 1.0000x reference)
import jax
import jax.numpy as jnp
from jax.experimental import pallas as pl
from jax.experimental.pallas import tpu as pltpu

B = 16384
D = 2176
HID = 64
BM = 512


def _mlp_body(x_ref, wh_ref, bh_ref, wo_ref, bo_ref, o_ref):
    wt = jax.lax.dot_general(
        wo_ref[...], wh_ref[...], (((0,), (1,)), ((), ())),
        preferred_element_type=jnp.float32,
    )
    c = jnp.dot(bh_ref[...], wo_ref[...], preferred_element_type=jnp.float32)
    z = jax.lax.dot_general(
        wt, x_ref[...], (((1,), (1,)), ((), ())),
        preferred_element_type=jnp.float32,
    )
    o_ref[...] = jax.nn.sigmoid(z + (c + bo_ref[...])).reshape(1, 1, BM)


def kernel(inputs, W_h, b_h, W_o, b_o):
    bh2 = b_h.reshape(1, HID)
    bo2 = b_o.reshape(1, 1)
    out = pl.pallas_call(
        _mlp_body,
        grid=(B // BM,),
        in_specs=[
            pl.BlockSpec((BM, D), lambda i: (i, 0)),
            pl.BlockSpec((D, HID), lambda i: (0, 0)),
            pl.BlockSpec((1, HID), lambda i: (0, 0)),
            pl.BlockSpec((HID, 1), lambda i: (0, 0)),
            pl.BlockSpec((1, 1), lambda i: (0, 0)),
        ],
        out_specs=pl.BlockSpec((1, 1, BM), lambda i: (i, 0, 0)),
        out_shape=jax.ShapeDtypeStruct((B // BM, 1, BM), jnp.float32),
        compiler_params=pltpu.CompilerParams(
            dimension_semantics=("arbitrary",),
        ),
    )(inputs, W_h, bh2, W_o, bo2)
    return out.reshape(B, 1)


# speedup vs baseline: 1.1609x; 1.1609x over previous
import jax
import jax.numpy as jnp
from jax.experimental import pallas as pl
from jax.experimental.pallas import tpu as pltpu

B = 16384
D = 2176
HID = 64
CM = 1024
NCHUNK = B // CM
NSLOT = 4


def _mlp_body(x_hbm, wh_ref, bh_ref, wo_ref, bo_ref, o_ref, x_vmem, sems):
    wt = jax.lax.dot_general(
        wo_ref[...], wh_ref[...], (((0,), (1,)), ((), ())),
        preferred_element_type=jnp.float32,
    )
    c = jnp.dot(bh_ref[...], wo_ref[...], preferred_element_type=jnp.float32)
    c = c + bo_ref[...]

    def copy(i, slot):
        return pltpu.make_async_copy(
            x_hbm.at[pl.ds(i * CM, CM), :],
            x_vmem.at[slot],
            sems.at[slot],
        )

    for s in range(NSLOT):
        copy(s, s).start()

    for i in range(NCHUNK):
        slot = i % NSLOT
        copy(i, slot).wait()
        z = jax.lax.dot_general(
            wt, x_vmem[slot], (((1,), (1,)), ((), ())),
            preferred_element_type=jnp.float32,
        )
        o_ref[i] = jax.nn.sigmoid(z + c)
        if i + NSLOT < NCHUNK:
            copy(i + NSLOT, slot).start()


def kernel(inputs, W_h, b_h, W_o, b_o):
    bh2 = b_h.reshape(1, HID)
    bo2 = b_o.reshape(1, 1)
    out = pl.pallas_call(
        _mlp_body,
        in_specs=[
            pl.BlockSpec(memory_space=pltpu.HBM),
            pl.BlockSpec(memory_space=pltpu.VMEM),
            pl.BlockSpec(memory_space=pltpu.VMEM),
            pl.BlockSpec(memory_space=pltpu.VMEM),
            pl.BlockSpec(memory_space=pltpu.VMEM),
        ],
        out_specs=pl.BlockSpec(memory_space=pltpu.VMEM),
        out_shape=jax.ShapeDtypeStruct((NCHUNK, 1, CM), jnp.float32),
        scratch_shapes=[
            pltpu.VMEM((NSLOT, CM, D), jnp.float32),
            pltpu.SemaphoreType.DMA((NSLOT,)),
        ],
    )(inputs, W_h, bh2, W_o, bo2)
    return out.reshape(B, 1)


# speedup vs baseline: 1.1777x; 1.0145x over previous
import jax
import jax.numpy as jnp
from jax.experimental import pallas as pl
from jax.experimental.pallas import tpu as pltpu

B = 16384
D = 2176
HID = 64
BM = 1024


def _mlp_body(x_ref, x2_ref, wh_ref, bh_ref, wo_ref, bo_ref, o_ref):
    wt = jax.lax.dot_general(
        wo_ref[...], wh_ref[...], (((0,), (1,)), ((), ())),
        preferred_element_type=jnp.float32,
    )
    c = jnp.dot(bh_ref[...], wo_ref[...], preferred_element_type=jnp.float32)
    cc = c + bo_ref[...]
    for s, xr in enumerate((x_ref, x2_ref)):
        z = jax.lax.dot_general(
            wt, xr[...], (((1,), (1,)), ((), ())),
            preferred_element_type=jnp.float32,
        )
        o_ref[s, 0] = jax.nn.sigmoid(z + cc)


def kernel(inputs, W_h, b_h, W_o, b_o):
    bh2 = b_h.reshape(1, HID)
    bo2 = b_o.reshape(1, 1)
    nbs = B // BM // 2
    out = pl.pallas_call(
        _mlp_body,
        grid=(nbs,),
        in_specs=[
            pl.BlockSpec((BM, D), lambda i: (i, 0)),
            pl.BlockSpec((BM, D), lambda i, _n=nbs: (i + _n, 0)),
            pl.BlockSpec((D, HID), lambda i: (0, 0)),
            pl.BlockSpec((1, HID), lambda i: (0, 0)),
            pl.BlockSpec((HID, 1), lambda i: (0, 0)),
            pl.BlockSpec((1, 1), lambda i: (0, 0)),
        ],
        out_specs=pl.BlockSpec((2, 1, 1, BM), lambda i: (0, i, 0, 0)),
        out_shape=jax.ShapeDtypeStruct((2, B // BM // 2, 1, BM), jnp.float32),
        compiler_params=pltpu.CompilerParams(
            dimension_semantics=("arbitrary",),
        ),
    )(inputs, inputs, W_h, bh2, W_o, bo2)
    return out.reshape(B, 1)


# speedup vs baseline: 1.2177x; 1.0339x over previous
import jax
import jax.numpy as jnp
from jax.experimental import pallas as pl
from jax.experimental.pallas import tpu as pltpu

B = 16384
D = 2176
HID = 64
BM = 1024


def _mlp_body(x_ref, wh_ref, bh_ref, wo_ref, bo_ref, o_ref, wt_ref, c_ref):
    @pl.when(pl.program_id(0) == 0)
    def _fold_weights():
        wt_ref[...] = jax.lax.dot_general(
            wo_ref[...], wh_ref[...], (((0,), (1,)), ((), ())),
            preferred_element_type=jnp.float32,
        )
        c_ref[...] = (
            jnp.dot(bh_ref[...], wo_ref[...], preferred_element_type=jnp.float32)
            + bo_ref[...]
        )

    z = jax.lax.dot_general(
        wt_ref[...], x_ref[...], (((1,), (1,)), ((), ())),
        preferred_element_type=jnp.float32,
    )
    o_ref[...] = jax.nn.sigmoid(z + c_ref[...]).reshape(1, 1, BM)


def kernel(inputs, W_h, b_h, W_o, b_o):
    bh2 = b_h.reshape(1, HID)
    bo2 = b_o.reshape(1, 1)
    out = pl.pallas_call(
        _mlp_body,
        grid=(B // BM,),
        in_specs=[
            pl.BlockSpec((BM, D), lambda i: (i, 0)),
            pl.BlockSpec((D, HID), lambda i: (0, 0)),
            pl.BlockSpec((1, HID), lambda i: (0, 0)),
            pl.BlockSpec((HID, 1), lambda i: (0, 0)),
            pl.BlockSpec((1, 1), lambda i: (0, 0)),
        ],
        out_specs=pl.BlockSpec((1, 1, BM), lambda i: (i, 0, 0)),
        out_shape=jax.ShapeDtypeStruct((B // BM, 1, BM), jnp.float32),
        scratch_shapes=[
            pltpu.VMEM((1, D), jnp.float32),
            pltpu.VMEM((1, 1), jnp.float32),
        ],
        compiler_params=pltpu.CompilerParams(
            dimension_semantics=("arbitrary",),
        ),
    )(inputs, W_h, bh2, W_o, bo2)
    return out.reshape(B, 1)
